# baseline (speedup 1.0000x reference)
import functools

import jax
import jax.numpy as jnp
from jax import lax
from jax.experimental import pallas as pl
from jax.experimental.pallas import tpu as pltpu
from jax.experimental.pallas import tpu_sc as plsc

VOCAB = 50257
HIDDEN = 768
MAX_POS = 2048
BATCH = 4
SEQ = 2048
TOK = BATCH * SEQ
LANES = 16
NC, NS = 2, 16
NW = NC * NS
TPW = TOK // NW
PPW = SEQ // NW
CH = 16
NPC = PPW // CH
NCH = NPC * BATCH
HV = HIDDEN // LANES
NVB = 6
NWP = 2
NSH = 2

_mesh = plsc.VectorSubcoreMesh(core_axis_name="c", subcore_axis_name="s")


@functools.partial(
    pl.kernel,
    mesh=_mesh,
    out_type=jax.ShapeDtypeStruct((TOK, HIDDEN), jnp.float32),
    scratch_types=[
        pltpu.VMEM((TPW,), jnp.int32),
        pltpu.VMEM((CH, HIDDEN), jnp.float32),
        pltpu.VMEM((CH, HIDDEN), jnp.float32),
        pltpu.VMEM((CH, HIDDEN), jnp.float32),
        pltpu.VMEM((CH, HIDDEN), jnp.float32),
        pltpu.VMEM((CH, HIDDEN), jnp.float32),
        pltpu.VMEM((CH, HIDDEN), jnp.float32),
        pltpu.VMEM((CH, HIDDEN), jnp.float32),
        pltpu.VMEM((CH, HIDDEN), jnp.float32),
        pltpu.VMEM_SHARED((NS * CH, HIDDEN), jnp.float32),
        pltpu.VMEM_SHARED((NS * CH, HIDDEN), jnp.float32),
        pltpu.SemaphoreType.DMA,
        pltpu.SemaphoreType.DMA,
        pltpu.SemaphoreType.DMA,
        pltpu.SemaphoreType.DMA,
        pltpu.SemaphoreType.DMA,
        pltpu.SemaphoreType.DMA,
        pltpu.SemaphoreType.DMA,
        pltpu.SemaphoreType.DMA,
        pltpu.SemaphoreType.DMA,
        pltpu.SemaphoreType.DMA,
        pltpu.SemaphoreType.DMA,
        pltpu.SemaphoreType.DMA,
        pltpu.SemaphoreType.DMA,
    ],
)
def _embed(ids_hbm, wte_hbm, wpe_hbm, out_hbm, idx_v,
           vb0, vb1, vb2, vb3, vb4, vb5, wp0, wp1, sh0, sh1,
           g0, g1, g2, g3, g4, g5, w0, w1, x0, x1, o0, o1, isem):
    cid = lax.axis_index("c")
    sid = lax.axis_index("s")
    wid = sid * NC + cid
    pos0 = wid * PPW
    row0 = sid * CH
    vbufs = [vb0, vb1, vb2, vb3, vb4, vb5]
    wbufs = [wp0, wp1]
    shs = [sh0, sh1]
    gsems = [g0, g1, g2, g3, g4, g5]
    wsems = [w0, w1]
    xsems = [x0, x1]
    osems = [o0, o1]

    did = [
        pltpu.async_copy(
            ids_hbm.at[b, pl.ds(pos0 + q * CH, CH)],
            idx_v.at[pl.ds((q * BATCH + b) * CH, CH)], isem)
        for q in range(NPC) for b in range(BATCH)
    ]
    for d in did:
        d.wait()

    def gath(k):
        r = k % NVB
        return pltpu.async_copy(
            wte_hbm.at[idx_v.at[pl.ds(k * CH, CH)]], vbufs[r], gsems[r])

    def wpe(q):
        r = q % NWP
        return pltpu.async_copy(
            wpe_hbm.at[pl.ds(pos0 + q * CH, CH)], wbufs[r], wsems[r])

    def xbar(k):
        r = k % NSH
        return pltpu.async_copy(
            vbufs[k % NVB], shs[r].at[pl.ds(row0, CH)], xsems[r])

    def hbm_wb(k):
        q, b = divmod(k, BATCH)
        r = k % NSH
        return pltpu.async_copy(
            shs[r].at[pl.ds(row0, CH)],
            out_hbm.at[pl.ds(b * SEQ + pos0 + q * CH, CH)], osems[r])

    dg = {j: gath(j) for j in range(4)}
    dwp = {0: wpe(0), 1: wpe(1)}
    dx = {}
    dw2 = {}
    for k in range(NCH):
        q, b = divmod(k, BATCH)
        if k >= 1:
            dx[k - 1].wait()
            dw2[k - 1] = hbm_wb(k - 1)
        if k >= 2:
            dw2[k - 2].wait()
        if k + 4 < NCH:
            dg[k + 4] = gath(k + 4)
        if b == 0:
            dwp[q].wait()
        dg[k].wait()

        rows = vbufs[k % NVB]
        wrows = wbufs[q % NWP]

        def add_row(i, carry):
            for j in range(HV):
                s = pl.ds(j * LANES, LANES)
                plsc.addupdate(rows.at[i, s], wrows[i, s])
            return carry

        lax.fori_loop(0, CH, add_row, 0)
        dx[k] = xbar(k)
        if b == BATCH - 1 and q + 2 < NPC:
            dwp[q + 2] = wpe(q + 2)
    dx[NCH - 1].wait()
    dw2[NCH - 1] = hbm_wb(NCH - 1)
    dw2[NCH - 2].wait()
    dw2[NCH - 1].wait()


def kernel(input_ids, wte, wpe):
    ids = input_ids.astype(jnp.int32)
    out = _embed(ids, wte, wpe)
    return out.reshape(BATCH, SEQ, HIDDEN)


# speedup vs baseline: 1.1385x; 1.1385x over previous
import functools

import jax
import jax.numpy as jnp
from jax import lax
from jax.experimental import pallas as pl
from jax.experimental.pallas import tpu as pltpu
from jax.experimental.pallas import tpu_sc as plsc

VOCAB = 50257
HIDDEN = 768
MAX_POS = 2048
BATCH = 4
SEQ = 2048
TOK = BATCH * SEQ
LANES = 16
NC, NS = 2, 16
NW = NC * NS
TPW = TOK // NW
PPW = SEQ // NW
CH = 32
NPC = PPW // CH
NCH = NPC * BATCH
HV = HIDDEN // LANES
NVB = 3
NWP = 2

_mesh = plsc.VectorSubcoreMesh(core_axis_name="c", subcore_axis_name="s")


@functools.partial(
    pl.kernel,
    mesh=_mesh,
    out_type=jax.ShapeDtypeStruct((TOK, HIDDEN), jnp.float32),
    scratch_types=[
        pltpu.VMEM((TPW,), jnp.int32),
        pltpu.VMEM((CH, HIDDEN), jnp.float32),
        pltpu.VMEM((CH, HIDDEN), jnp.float32),
        pltpu.VMEM((CH, HIDDEN), jnp.float32),
        pltpu.VMEM((CH, HIDDEN), jnp.float32),
        pltpu.VMEM((CH, HIDDEN), jnp.float32),
        pltpu.SemaphoreType.DMA,
        pltpu.SemaphoreType.DMA,
        pltpu.SemaphoreType.DMA,
        pltpu.SemaphoreType.DMA,
        pltpu.SemaphoreType.DMA,
        pltpu.SemaphoreType.DMA,
        pltpu.SemaphoreType.DMA,
        pltpu.SemaphoreType.DMA,
        pltpu.SemaphoreType.DMA,
    ],
)
def _embed(ids_hbm, wte_hbm, wpe_hbm, out_hbm, idx_v,
           vb0, vb1, vb2, wp0, wp1,
           g0, g1, g2, w0, w1, o0, o1, o2, isem):
    cid = lax.axis_index("c")
    sid = lax.axis_index("s")
    wid = sid * NC + cid
    pos0 = wid * PPW
    vbufs = [vb0, vb1, vb2]
    wbufs = [wp0, wp1]
    gsems = [g0, g1, g2]
    wsems = [w0, w1]
    osems = [o0, o1, o2]

    did = [
        pltpu.async_copy(
            ids_hbm.at[b, pl.ds(pos0 + q * CH, CH)],
            idx_v.at[pl.ds((q * BATCH + b) * CH, CH)], isem)
        for q in range(NPC) for b in range(BATCH)
    ]
    for d in did:
        d.wait()

    def gath(k):
        r = k % NVB
        return pltpu.async_copy(
            wte_hbm.at[idx_v.at[pl.ds(k * CH, CH)]], vbufs[r], gsems[r])

    def wpe(q):
        r = q % NWP
        return pltpu.async_copy(
            wpe_hbm.at[pl.ds(pos0 + q * CH, CH)], wbufs[r], wsems[r])

    def wb(k):
        q, b = divmod(k, BATCH)
        r = k % NVB
        return pltpu.async_copy(
            vbufs[r], out_hbm.at[pl.ds(b * SEQ + pos0 + q * CH, CH)],
            osems[r])

    dg = {j: gath(j) for j in range(2)}
    dwp = {0: wpe(0), 1: wpe(1)}
    dw = {}
    for k in range(NCH):
        q, b = divmod(k, BATCH)
        if k + 2 < NCH:
            if k - 1 >= 0:
                dw[k - 1].wait()
            dg[k + 2] = gath(k + 2)
        if b == 0:
            dwp[q].wait()
        dg[k].wait()

        rows = vbufs[k % NVB]
        wrows = wbufs[q % NWP]

        def add_row(i, carry):
            for j in range(HV):
                s = pl.ds(j * LANES, LANES)
                plsc.addupdate(rows.at[i, s], wrows[i, s])
            return carry

        lax.fori_loop(0, CH, add_row, 0)
        dw[k] = wb(k)
        if b == BATCH - 1 and q + 2 < NPC:
            dwp[q + 2] = wpe(q + 2)
    for j in range(NCH - 3, NCH):
        dw[j].wait()


def kernel(input_ids, wte, wpe):
    ids = input_ids.astype(jnp.int32)
    out = _embed(ids, wte, wpe)
    return out.reshape(BATCH, SEQ, HIDDEN)
